# Initial kernel scaffold; baseline (speedup 1.0000x reference)
import jax
import jax.numpy as jnp
from jax.experimental import pallas as pl


def kernel(pos_edge_index, neg_edge_index, pos_weight, neg_weight, x_emb, c1_Wp, c1_bp, c1_Wn, c1_bn, c2_Wp, c2_bp, c2_Wn, c2_bn, W_out, b_out):
    raise NotImplementedError("write your pallas kernel here")


# speedup vs baseline: 7.3187x; 7.3187x over previous
import functools

import jax
import jax.numpy as jnp
from jax import lax
from jax.experimental import pallas as pl
from jax.experimental.pallas import tpu as pltpu
from jax.experimental.pallas import tpu_sc as plsc

N = 50000
IN_DIM = 64
H = 32
E_POS = 600000
E_NEG = 200000
E_TOT = E_POS + E_NEG

NTILE = 16
CHUNK = 512
KCH = -(-(E_TOT // NTILE) // CHUNK)
P_EDGE = NTILE * KCH * CHUNK

BN = 2000
f32 = jnp.float32


def _t1_body(x_ref, w_ref, b_ref, o_ref):
    o_ref[...] = jnp.dot(x_ref[...], w_ref[...],
                         preferred_element_type=f32) + b_ref[...]


def _t2_body(s_ref, a_ref, w_ref, b_ref, o_ref):
    z = jnp.tanh(s_ref[...] + a_ref[...])
    o_ref[...] = jnp.dot(z, w_ref[...], preferred_element_type=f32) + b_ref[...]


def _t3_body(s_ref, a_ref, w_ref, b_ref, o_ref):
    z = jnp.tanh(s_ref[...] + a_ref[...])
    o_ref[...] = jnp.tanh(jnp.dot(z, w_ref[...],
                                  preferred_element_type=f32) + b_ref[...])


def _row_mm(body, x, a, w, b):
    n, kin = x.shape
    kout = w.shape[1]
    grid = (n // BN,)
    args = [x] + ([a] if a is not None else []) + [w, b]
    in_specs = [pl.BlockSpec((BN, kin), lambda i: (i, 0))]
    if a is not None:
        in_specs.append(pl.BlockSpec((BN, kin), lambda i: (i, 0)))
    in_specs += [pl.BlockSpec((kin, kout), lambda i: (0, 0)),
                 pl.BlockSpec((1, kout), lambda i: (0, 0))]
    return pl.pallas_call(
        body,
        grid=grid,
        in_specs=in_specs,
        out_specs=pl.BlockSpec((BN, kout), lambda i: (i, 0)),
        out_shape=jax.ShapeDtypeStruct((n, kout), f32),
    )(*args)


def _make_segsum(nacc, d):
    nrt = -(-(nacc // NTILE) // 8) * 8
    nrt_last = nacc - (NTILE - 1) * nrt
    assert nrt_last > 0 and nrt_last % 8 == 0

    mesh = plsc.VectorSubcoreMesh(core_axis_name="c", subcore_axis_name="s")

    @functools.partial(
        pl.kernel,
        out_type=jax.ShapeDtypeStruct((2 * nacc, d), f32),
        mesh=mesh,
        compiler_params=pltpu.CompilerParams(use_tc_tiling_on_sc=False),
        scratch_types=[
            pltpu.VMEM_SHARED((nacc, d), f32),
            pltpu.VMEM((CHUNK,), jnp.int32),
            pltpu.VMEM((CHUNK,), jnp.int32),
            pltpu.VMEM((CHUNK,), f32),
            pltpu.VMEM((CHUNK, d), f32),
            pltpu.SemaphoreType.DMA,
        ],
    )
    def seg(tab0, tab1, src, dst, w, zrows, out,
            acc, src_v, dst_v, w_v, rows_v, sem):
        cid = lax.axis_index("c")
        sid = lax.axis_index("s")
        @pl.when(sid < NTILE - 1)
        def _():
            pltpu.sync_copy(zrows.at[pl.ds(sid * nrt, nrt)],
                            acc.at[pl.ds(sid * nrt, nrt)])

        @pl.when(sid == NTILE - 1)
        def _():
            pltpu.sync_copy(zrows.at[pl.ds(sid * nrt, nrt_last)],
                            acc.at[pl.ds(sid * nrt, nrt_last)])

        plsc.subcore_barrier()

        base = sid * (KCH * CHUNK)

        def run(tab):
            def chunk(kk, carry):
                off = base + kk * CHUNK
                pltpu.sync_copy(src.at[pl.ds(off, CHUNK)], src_v)
                pltpu.sync_copy(w.at[pl.ds(off, CHUNK)], w_v)
                pltpu.sync_copy(dst.at[pl.ds(off, CHUNK)], dst_v)
                pltpu.async_copy(tab.at[src_v], rows_v, sem).wait()

                def scale16(g, c2):
                    wv = w_v[pl.ds(g * 16, 16)]
                    for j in range(16):
                        wb = wv[j]
                        e = g * 16 + j
                        for dd in range(d // 16):
                            sl = pl.ds(dd * 16, 16)
                            rows_v[e, sl] = rows_v[e, sl] * wb
                    return c2

                lax.fori_loop(0, CHUNK // 16, scale16, 0)
                pltpu.sync_copy(rows_v, acc.at[dst_v], add=True)
                return carry

            lax.fori_loop(0, KCH, chunk, 0)

        @pl.when(cid == 0)
        def _():
            run(tab0)

        @pl.when(cid == 1)
        def _():
            run(tab1)

        plsc.subcore_barrier()

        @pl.when(sid < NTILE - 1)
        def _():
            pltpu.sync_copy(acc.at[pl.ds(sid * nrt, nrt)],
                            out.at[pl.ds(cid * nacc + sid * nrt, nrt)])

        @pl.when(sid == NTILE - 1)
        def _():
            pltpu.sync_copy(acc.at[pl.ds(sid * nrt, nrt_last)],
                            out.at[pl.ds(cid * nacc + sid * nrt, nrt_last)])

    return seg


_seg1 = _make_segsum(2 * N, 16)
_seg2 = _make_segsum(N, 32)


def kernel(pos_edge_index, neg_edge_index, pos_weight, neg_weight, x_emb,
           c1_Wp, c1_bp, c1_Wn, c1_bn, c2_Wp, c2_bp, c2_Wn, c2_bn,
           W_out, b_out):
    ps, pd = pos_edge_index[0], pos_edge_index[1]
    ns, nd = neg_edge_index[0], neg_edge_index[1]
    pad = P_EDGE - E_TOT
    zpad_i = jnp.zeros((pad,), jnp.int32)
    src = jnp.concatenate([ps, ns + N, zpad_i])
    dst1 = jnp.concatenate([pd, nd + N, zpad_i])
    dst2 = jnp.concatenate([pd, nd, zpad_i])
    w = jnp.concatenate([pos_weight, neg_weight, jnp.zeros((pad,), f32)])

    W1 = jnp.concatenate([c1_Wp[:IN_DIM], c1_Wn[:IN_DIM],
                          c1_Wp[IN_DIM:], c1_Wn[IN_DIM:]], axis=1)
    b1 = jnp.concatenate([jnp.zeros((64,), f32), c1_bp, c1_bn]).reshape(1, 128)
    Z = jnp.zeros((H, H), f32)
    W2 = jnp.concatenate([
        jnp.concatenate([c2_Wp[0:H], Z, Z, c2_Wn[H:2 * H],
                         c2_Wp[2 * H:], Z], axis=1),
        jnp.concatenate([Z, c2_Wn[0:H], c2_Wp[H:2 * H], Z,
                         Z, c2_Wn[2 * H:]], axis=1),
    ], axis=0)
    b2 = jnp.concatenate([jnp.zeros((128,), f32), c2_bp, c2_bn]).reshape(1, 192)

    y1 = _row_mm(_t1_body, x_emb, None, W1, b1)
    xp, xn, s1 = y1[:, :H], y1[:, H:2 * H], y1[:, 2 * H:]
    tab1_0 = jnp.concatenate([xp[:, :16], xn[:, :16]], axis=0)
    tab1_1 = jnp.concatenate([xp[:, 16:], xn[:, 16:]], axis=0)

    zeros1 = jnp.zeros((2 * N, 16), f32)
    agg1 = _seg1(tab1_0, tab1_1, src, dst1, w, zeros1)
    aggcat = jnp.concatenate(
        [agg1[0:N], agg1[2 * N:3 * N],
         agg1[N:2 * N], agg1[3 * N:4 * N]], axis=1)

    y2 = _row_mm(_t2_body, s1, aggcat, W2, b2)
    up, un, s2 = y2[:, 0:64], y2[:, 64:128], y2[:, 128:192]
    tab2_0 = jnp.concatenate([up[:, :H], un[:, :H]], axis=0)
    tab2_1 = jnp.concatenate([up[:, H:], un[:, H:]], axis=0)

    zeros2 = jnp.zeros((N, 32), f32)
    agg2 = _seg2(tab2_0, tab2_1, src, dst2, w, zeros2)
    a2 = jnp.concatenate([agg2[0:N], agg2[N:2 * N]], axis=1)

    return _row_mm(_t3_body, s2, a2, W_out, b_out.reshape(1, IN_DIM))


# speedup vs baseline: 9.8435x; 1.3450x over previous
import functools

import jax
import jax.numpy as jnp
from jax import lax
from jax.experimental import pallas as pl
from jax.experimental.pallas import tpu as pltpu
from jax.experimental.pallas import tpu_sc as plsc

N = 50000
IN_DIM = 64
H = 32
E_POS = 600000
E_NEG = 200000

NTILE = 16
CHUNK = 512

BN = 2000
f32 = jnp.float32


def _nchunks(e):
    return -(-e // CHUNK)


def _t1_body(x_ref, w_ref, b_ref, xp0, xp1, xn0, xn1, s1):
    y = jnp.dot(x_ref[...], w_ref[...], preferred_element_type=f32) + b_ref[...]
    xp0[...] = y[:, 0:16]
    xp1[...] = y[:, 16:32]
    xn0[...] = y[:, 32:48]
    xn1[...] = y[:, 48:64]
    s1[...] = y[:, 64:128]


def _t2_body(s_ref, ap0, ap1, an0, an1, w_ref, b_ref,
             up0, up1, un0, un1, s2):
    agg = jnp.concatenate([ap0[...], ap1[...], an0[...], an1[...]], axis=1)
    z = jnp.tanh(s_ref[...] + agg)
    y = jnp.dot(z, w_ref[...], preferred_element_type=f32) + b_ref[...]
    up0[...] = y[:, 0:32]
    up1[...] = y[:, 32:64]
    un0[...] = y[:, 64:96]
    un1[...] = y[:, 96:128]
    s2[...] = y[:, 128:192]


def _t3_body(s_ref, a0, a1, w_ref, b_ref, o_ref):
    z = jnp.tanh(s_ref[...] + jnp.concatenate([a0[...], a1[...]], axis=1))
    o_ref[...] = jnp.tanh(jnp.dot(z, w_ref[...],
                                  preferred_element_type=f32) + b_ref[...])


def _blk(c, i_map=lambda i: (i, 0)):
    return pl.BlockSpec((BN, c), i_map)


def _wblk(r, c):
    return pl.BlockSpec((r, c), lambda i: (0, 0))


_GRID = (N // BN,)

_t1 = functools.partial(
    pl.pallas_call, _t1_body, grid=_GRID,
    in_specs=[_blk(64), _wblk(64, 128), _wblk(1, 128)],
    out_specs=[_blk(16), _blk(16), _blk(16), _blk(16), _blk(64)],
    out_shape=[jax.ShapeDtypeStruct((N, 16), f32)] * 4
    + [jax.ShapeDtypeStruct((N, 64), f32)],
)()

_t2 = functools.partial(
    pl.pallas_call, _t2_body, grid=_GRID,
    in_specs=[_blk(64), _blk(16), _blk(16), _blk(16), _blk(16),
              _wblk(64, 192), _wblk(1, 192)],
    out_specs=[_blk(32), _blk(32), _blk(32), _blk(32), _blk(64)],
    out_shape=[jax.ShapeDtypeStruct((N, 32), f32)] * 4
    + [jax.ShapeDtypeStruct((N, 64), f32)],
)()

_t3 = functools.partial(
    pl.pallas_call, _t3_body, grid=_GRID,
    in_specs=[_blk(64), _blk(32), _blk(32), _wblk(64, 64), _wblk(1, 64)],
    out_specs=_blk(64),
    out_shape=jax.ShapeDtypeStruct((N, 64), f32),
)()


def _make_segsum(d, separate):
    kp, kn = _nchunks(E_POS), _nchunks(E_NEG)
    ovr_p, ovr_n = kp * CHUNK - E_POS, kn * CHUNK - E_NEG
    nrt = -(-(N // NTILE) // 8) * 8
    nrt_last = N - (NTILE - 1) * nrt
    assert nrt_last > 0 and nrt_last % 8 == 0
    assert ovr_p % 16 == 0 and ovr_n % 16 == 0

    mesh = plsc.VectorSubcoreMesh(core_axis_name="c", subcore_axis_name="s")
    nout = 4 if separate else 2

    @functools.partial(
        pl.kernel,
        out_type=[jax.ShapeDtypeStruct((N, d), f32)] * nout,
        mesh=mesh,
        compiler_params=pltpu.CompilerParams(use_tc_tiling_on_sc=False),
        scratch_types=[pltpu.VMEM_SHARED((N, d), f32)] * (2 if separate else 1)
        + [
            pltpu.VMEM((CHUNK,), jnp.int32),
            pltpu.VMEM((CHUNK,), jnp.int32),
            pltpu.VMEM((CHUNK,), f32),
            pltpu.VMEM((CHUNK, d), f32),
            pltpu.SemaphoreType.DMA,
        ],
    )
    def seg(tp0, tp1, tn0, tn1, pos_ei, neg_ei, pos_w, neg_w, zrows,
            *out_and_scratch):
        outs = out_and_scratch[:nout]
        if separate:
            acc_p, acc_n = out_and_scratch[nout:nout + 2]
            src_v, dst_v, w_v, rows_v, sem = out_and_scratch[nout + 2:]
        else:
            acc_p = acc_n = out_and_scratch[nout]
            src_v, dst_v, w_v, rows_v, sem = out_and_scratch[nout + 1:]
        cid = lax.axis_index("c")
        sid = lax.axis_index("s")

        def rowslice(ref, other=None):
            tgt = ref if other is None else other

            @pl.when(sid < NTILE - 1)
            def _():
                pltpu.sync_copy(ref.at[pl.ds(sid * nrt, nrt)],
                                tgt.at[pl.ds(sid * nrt, nrt)])

            @pl.when(sid == NTILE - 1)
            def _():
                pltpu.sync_copy(ref.at[pl.ds(sid * nrt, nrt_last)],
                                tgt.at[pl.ds(sid * nrt, nrt_last)])

        rowslice(zrows, acc_p)
        if separate:
            rowslice(zrows, acc_n)
        plsc.subcore_barrier()

        def do_chunk(ei, wts, tab, acc, off, nz):
            pltpu.sync_copy(ei.at[0, pl.ds(off, CHUNK)], src_v)
            pltpu.sync_copy(ei.at[1, pl.ds(off, CHUNK)], dst_v)
            pltpu.sync_copy(wts.at[pl.ds(off, CHUNK)], w_v)
            for g in range(nz // 16):
                w_v[pl.ds(g * 16, 16)] = jnp.zeros((16,), f32)
            pltpu.async_copy(tab.at[src_v], rows_v, sem).wait()

            def scale16(g, c2):
                wv = w_v[pl.ds(g * 16, 16)]
                for j in range(16):
                    wb = wv[j]
                    e = g * 16 + j
                    for dd in range(d // 16):
                        sl = pl.ds(dd * 16, 16)
                        rows_v[e, sl] = rows_v[e, sl] * wb
                return c2

            lax.fori_loop(0, CHUNK // 16, scale16, 0)
            pltpu.sync_copy(rows_v, acc.at[dst_v], add=True)

        def run_edges(ei, wts, e_tot, ktot, ovr, tab, acc):
            nreg = ((ktot - 1) - sid + NTILE - 1) // NTILE

            def chunk(jj, carry):
                do_chunk(ei, wts, tab, acc, (sid + jj * NTILE) * CHUNK, 0)
                return carry

            lax.fori_loop(0, nreg, chunk, 0)

            @pl.when(sid == (ktot - 1) % NTILE)
            def _():
                do_chunk(ei, wts, tab, acc, e_tot - CHUNK, ovr)

        def body(tp, tn):
            run_edges(pos_ei, pos_w, E_POS, kp, ovr_p, tp, acc_p)
            run_edges(neg_ei, neg_w, E_NEG, kn, ovr_n, tn, acc_n)

        @pl.when(cid == 0)
        def _():
            body(tp0, tn0)

        @pl.when(cid == 1)
        def _():
            body(tp1, tn1)

        plsc.subcore_barrier()
        if separate:
            accs = (acc_p, acc_n)
            for c in (0, 1):
                @pl.when(cid == c)
                def _(c=c):
                    rowslice(accs[0], outs[c])
                    rowslice(accs[1], outs[nout // 2 + c])
        else:
            for c in (0, 1):
                @pl.when(cid == c)
                def _(c=c):
                    rowslice(acc_p, outs[c])

    return seg


_seg1 = _make_segsum(16, True)
_seg2 = _make_segsum(32, False)


def kernel(pos_edge_index, neg_edge_index, pos_weight, neg_weight, x_emb,
           c1_Wp, c1_bp, c1_Wn, c1_bn, c2_Wp, c2_bp, c2_Wn, c2_bn,
           W_out, b_out):
    W1 = jnp.concatenate([c1_Wp[:IN_DIM], c1_Wn[:IN_DIM],
                          c1_Wp[IN_DIM:], c1_Wn[IN_DIM:]], axis=1)
    b1 = jnp.concatenate([jnp.zeros((64,), f32), c1_bp, c1_bn]).reshape(1, 128)
    Z = jnp.zeros((H, H), f32)
    W2 = jnp.concatenate([
        jnp.concatenate([c2_Wp[0:H], Z, Z, c2_Wn[H:2 * H],
                         c2_Wp[2 * H:], Z], axis=1),
        jnp.concatenate([Z, c2_Wn[0:H], c2_Wp[H:2 * H], Z,
                         Z, c2_Wn[2 * H:]], axis=1),
    ], axis=0)
    b2 = jnp.concatenate([jnp.zeros((128,), f32), c2_bp, c2_bn]).reshape(1, 192)

    xp0, xp1, xn0, xn1, s1 = _t1(x_emb, W1, b1)

    zrows1 = jnp.zeros((N, 16), f32)
    ap0, ap1, an0, an1 = _seg1(xp0, xp1, xn0, xn1,
                               pos_edge_index, neg_edge_index,
                               pos_weight, neg_weight, zrows1)

    up0, up1, un0, un1, s2 = _t2(s1, ap0, ap1, an0, an1, W2, b2)

    zrows2 = jnp.zeros((N, 32), f32)
    a0, a1 = _seg2(up0, up1, un0, un1,
                   pos_edge_index, neg_edge_index,
                   pos_weight, neg_weight, zrows2)

    return _t3(s2, a0, a1, W_out, b_out.reshape(1, IN_DIM))


# speedup vs baseline: 11.4755x; 1.1658x over previous
import functools

import jax
import jax.numpy as jnp
from jax import lax
from jax.experimental import pallas as pl
from jax.experimental.pallas import tpu as pltpu
from jax.experimental.pallas import tpu_sc as plsc

N = 50000
IN_DIM = 64
H = 32
E_POS = 600000
E_NEG = 200000

NTILE = 16
CHUNK = 512

BN = 2000
f32 = jnp.float32


def _nchunks(e):
    return -(-e // CHUNK)


def _t1_body(x_ref, w_ref, b_ref, o_ref):
    o_ref[...] = jnp.dot(x_ref[...], w_ref[...],
                         preferred_element_type=f32) + b_ref[...]


def _t2_body(s_ref, a_ref, w_ref, b_ref, o_ref):
    z = jnp.tanh(s_ref[:, 64:128] + a_ref[:, 0:64])
    o_ref[...] = jnp.dot(z, w_ref[...], preferred_element_type=f32) + b_ref[...]


def _t3_body(s_ref, a_ref, w_ref, b_ref, o_ref):
    z = jnp.tanh(s_ref[:, 128:192] + a_ref[:, 0:64])
    o_ref[...] = jnp.tanh(jnp.dot(z, w_ref[...],
                                  preferred_element_type=f32) + b_ref[...])


_GRID = (N // BN,)


def _blk(c):
    return pl.BlockSpec((BN, c), lambda i: (i, 0))


def _wblk(r, c):
    return pl.BlockSpec((r, c), lambda i: (0, 0))


_t1 = functools.partial(
    pl.pallas_call, _t1_body, grid=_GRID,
    in_specs=[_blk(64), _wblk(64, 128), _wblk(1, 128)],
    out_specs=pl.BlockSpec((BN, 128), lambda i: (i, 0)),
    out_shape=jax.ShapeDtypeStruct((N, 128), f32),
)()

_t2 = functools.partial(
    pl.pallas_call, _t2_body, grid=_GRID,
    in_specs=[_blk(128), _blk(128), _wblk(64, 256), _wblk(1, 256)],
    out_specs=pl.BlockSpec((BN, 256), lambda i: (i, 0)),
    out_shape=jax.ShapeDtypeStruct((N, 256), f32),
)()

_t3 = functools.partial(
    pl.pallas_call, _t3_body, grid=_GRID,
    in_specs=[_blk(256), _blk(128), _wblk(64, 64), _wblk(1, 64)],
    out_specs=pl.BlockSpec((BN, 64), lambda i: (i, 0)),
    out_shape=jax.ShapeDtypeStruct((N, 64), f32),
)()


def _make_segsum(d, separate):
    kp, kn = _nchunks(E_POS), _nchunks(E_NEG)
    ovr_p, ovr_n = kp * CHUNK - E_POS, kn * CHUNK - E_NEG
    nrt = -(-(N // NTILE) // 8) * 8
    nrt_last = N - (NTILE - 1) * nrt
    assert nrt_last > 0 and nrt_last % 8 == 0
    assert ovr_p % 16 == 0 and ovr_n % 16 == 0

    mesh = plsc.VectorSubcoreMesh(core_axis_name="c", subcore_axis_name="s")

    @functools.partial(
        pl.kernel,
        out_type=jax.ShapeDtypeStruct((N, 128), f32),
        mesh=mesh,
        compiler_params=pltpu.CompilerParams(use_tc_tiling_on_sc=False),
        scratch_types=[pltpu.VMEM_SHARED((N, d), f32)] * (2 if separate else 1)
        + [
            pltpu.VMEM((CHUNK,), jnp.int32),
            pltpu.VMEM((CHUNK,), jnp.int32),
            pltpu.VMEM((CHUNK,), f32),
            pltpu.VMEM((CHUNK, d), f32),
            pltpu.SemaphoreType.DMA,
        ],
    )
    def seg(tab, pos_src, pos_dst, neg_src, neg_dst, pos_w, neg_w, zrows,
            out, *scratch):
        if separate:
            acc_p, acc_n = scratch[0:2]
            src_v, dst_v, w_v, rows_v, sem = scratch[2:]
        else:
            acc_p = acc_n = scratch[0]
            src_v, dst_v, w_v, rows_v, sem = scratch[1:]
        cid = lax.axis_index("c")
        sid = lax.axis_index("s")

        def rowslice(ref, tgt, coff=None):
            def cp(r0, nr):
                s = ref.at[pl.ds(r0, nr)]
                t = (tgt.at[pl.ds(r0, nr)] if coff is None
                     else tgt.at[pl.ds(r0, nr), pl.ds(coff, d)])
                pltpu.sync_copy(s, t)

            @pl.when(sid < NTILE - 1)
            def _():
                cp(sid * nrt, nrt)

            @pl.when(sid == NTILE - 1)
            def _():
                cp(sid * nrt, nrt_last)

        rowslice(zrows, acc_p)
        if separate:
            rowslice(zrows, acc_n)
        plsc.subcore_barrier()

        def do_chunk(srcs, dsts, wts, cb, acc, off, nz):
            pltpu.sync_copy(srcs.at[pl.ds(off, CHUNK)], src_v)
            pltpu.sync_copy(dsts.at[pl.ds(off, CHUNK)], dst_v)
            pltpu.sync_copy(wts.at[pl.ds(off, CHUNK)], w_v)
            for g in range(nz // 16):
                w_v[pl.ds(g * 16, 16)] = jnp.zeros((16,), f32)

            def to_flat(g, c2):
                sl = pl.ds(g * 16, 16)
                src_v[sl] = src_v[sl] * 8 + cb
                return c2

            lax.fori_loop(0, CHUNK // 16, to_flat, 0)
            pltpu.async_copy(tab.at[src_v], rows_v, sem).wait()

            def scale16(g, c2):
                wv = w_v[pl.ds(g * 16, 16)]
                for j in range(16):
                    wb = wv[j]
                    e = g * 16 + j
                    for dd in range(d // 16):
                        sl = pl.ds(dd * 16, 16)
                        rows_v[e, sl] = rows_v[e, sl] * wb
                return c2

            lax.fori_loop(0, CHUNK // 16, scale16, 0)
            pltpu.sync_copy(rows_v, acc.at[dst_v], add=True)

        def run_edges(srcs, dsts, wts, e_tot, ktot, ovr, cb, acc):
            nreg = ((ktot - 1) - sid + NTILE - 1) // NTILE

            def chunk(jj, carry):
                do_chunk(srcs, dsts, wts, cb, acc,
                         (sid + jj * NTILE) * CHUNK, 0)
                return carry

            lax.fori_loop(0, nreg, chunk, 0)

            @pl.when(sid == (ktot - 1) % NTILE)
            def _():
                do_chunk(srcs, dsts, wts, cb, acc, e_tot - CHUNK, ovr)

        run_edges(pos_src, pos_dst, pos_w, E_POS, kp, ovr_p, cid, acc_p)
        run_edges(neg_src, neg_dst, neg_w, E_NEG, kn, ovr_n, 2 + cid, acc_n)

        plsc.subcore_barrier()
        for c in (0, 1):
            @pl.when(cid == c)
            def _(c=c):
                if separate:
                    rowslice(acc_p, out, c * d)
                    rowslice(acc_n, out, (2 + c) * d)
                else:
                    rowslice(acc_p, out, c * d)

    return seg


_seg1 = _make_segsum(16, True)
_seg2 = _make_segsum(32, False)


def kernel(pos_edge_index, neg_edge_index, pos_weight, neg_weight, x_emb,
           c1_Wp, c1_bp, c1_Wn, c1_bn, c2_Wp, c2_bp, c2_Wn, c2_bn,
           W_out, b_out):
    ps, pd = pos_edge_index[0], pos_edge_index[1]
    ns, nd = neg_edge_index[0], neg_edge_index[1]

    W1 = jnp.concatenate([c1_Wp[:IN_DIM], c1_Wn[:IN_DIM],
                          c1_Wp[IN_DIM:], c1_Wn[IN_DIM:]], axis=1)
    b1 = jnp.concatenate([jnp.zeros((64,), f32), c1_bp, c1_bn]).reshape(1, 128)
    Z = jnp.zeros((H, H), f32)
    W2 = jnp.concatenate([
        jnp.concatenate([c2_Wp[0:H], Z, Z, c2_Wn[H:2 * H],
                         c2_Wp[2 * H:], Z, Z, Z], axis=1),
        jnp.concatenate([Z, c2_Wn[0:H], c2_Wp[H:2 * H], Z,
                         Z, c2_Wn[2 * H:], Z, Z], axis=1),
    ], axis=0)
    b2 = jnp.concatenate([jnp.zeros((128,), f32), c2_bp, c2_bn,
                          jnp.zeros((64,), f32)]).reshape(1, 256)

    y1 = _t1(x_emb, W1, b1)

    zrows1 = jnp.zeros((N, 16), f32)
    agg1 = _seg1(jnp.reshape(y1, (8 * N, 16)),
                 ps, pd, ns, nd, pos_weight, neg_weight, zrows1)

    y2 = _t2(y1, agg1, W2, b2)

    zrows2 = jnp.zeros((N, 32), f32)
    agg2 = _seg2(jnp.reshape(y2, (8 * N, 32)),
                 ps, pd, ns, nd, pos_weight, neg_weight, zrows2)

    return _t3(y2, agg2, W_out, b_out.reshape(1, IN_DIM))
